# baseline (speedup 1.0000x reference)
import functools

import jax
import jax.numpy as jnp
import numpy as np
from jax.experimental import pallas as pl
from jax.experimental.pallas import tpu as pltpu

DIM = 768
E = 64
K = 8
FILTER_RADIO = 0.62
LOAD_LR = 0.001
N = 32768
T = 512
C = 256


def _threefry2x32(ks0, ks1, x0, x1):
    rot_a = (13, 15, 26, 6)
    rot_b = (17, 29, 16, 24)
    ks2 = np.uint32(ks0 ^ ks1 ^ np.uint32(0x1BD11BDA))
    ks = (np.uint32(ks0), np.uint32(ks1), ks2)
    x0 = (x0 + ks[0]).astype(np.uint32)
    x1 = (x1 + ks[1]).astype(np.uint32)

    def rotl(v, d):
        return ((v << np.uint32(d)) | (v >> np.uint32(32 - d))).astype(np.uint32)

    def rounds(x0, x1, rots):
        for r in rots:
            x0 = (x0 + x1).astype(np.uint32)
            x1 = x0 ^ rotl(x1, r)
        return x0, x1

    x0, x1 = rounds(x0, x1, rot_a)
    x0 = (x0 + ks[1]).astype(np.uint32)
    x1 = (x1 + ks[2] + np.uint32(1)).astype(np.uint32)
    x0, x1 = rounds(x0, x1, rot_b)
    x0 = (x0 + ks[2]).astype(np.uint32)
    x1 = (x1 + ks[0] + np.uint32(2)).astype(np.uint32)
    x0, x1 = rounds(x0, x1, rot_a)
    x0 = (x0 + ks[0]).astype(np.uint32)
    x1 = (x1 + ks[1] + np.uint32(3)).astype(np.uint32)
    x0, x1 = rounds(x0, x1, rot_b)
    x0 = (x0 + ks[1]).astype(np.uint32)
    x1 = (x1 + ks[2] + np.uint32(4)).astype(np.uint32)
    x0, x1 = rounds(x0, x1, rot_a)
    x0 = (x0 + ks[2]).astype(np.uint32)
    x1 = (x1 + ks[0] + np.uint32(5)).astype(np.uint32)
    return x0, x1


def _compute_mask_const() -> np.ndarray:
    with np.errstate(over="ignore"):
        k0, k1 = _threefry2x32(np.uint32(0), np.uint32(0),
                               np.uint32(0), np.uint32(123))
    size = N * K
    counts = np.arange(size, dtype=np.uint32)
    b0, b1 = _threefry2x32(k0, k1, np.zeros(size, dtype=np.uint32), counts)
    bits = b0 ^ b1
    fl = ((bits >> np.uint32(9)) | np.uint32(0x3F800000)).view(np.float32)
    u = np.maximum(np.float32(0.0), fl - np.float32(1.0))
    return (u.reshape(N, K) > np.float32(FILTER_RADIO)).astype(np.float32)


_MASK_NP = _compute_mask_const()


def _router_kernel(x_ref, W_ref, b_ref, bi_ref, mask_ref,
                   out_ref, idx_ref, nbi_ref, cnt_ref):
    i = pl.program_id(0)
    nsteps = pl.num_programs(0)

    rows = jax.lax.broadcasted_iota(jnp.int32, (E, C), 0)
    selacc = jnp.zeros((E, C), jnp.float32)
    for c in range(T // C):
        lo = c * C
        logits_c = jax.lax.dot_general(
            W_ref[...], x_ref[lo:lo + C, :], (((1,), (1,)), ((), ())),
            preferred_element_type=jnp.float32)
        logits_c = logits_c + b_ref[...]
        work = logits_c + bi_ref[...]
        vals = []
        idxs = []
        for _ in range(K):
            m = jnp.max(work, axis=0, keepdims=True)
            eq = work == m
            idx = jnp.min(jnp.where(eq, rows, E), axis=0,
                          keepdims=True)
            onehot = rows == idx
            vals.append(jnp.sum(jnp.where(onehot, logits_c, 0.0), axis=0,
                                keepdims=True))
            idxs.append(idx)
            work = jnp.where(onehot, -jnp.inf, work)

        valsT = jnp.concatenate(vals, axis=0)
        idxT = jnp.concatenate(idxs, axis=0)

        mx = jnp.max(valsT, axis=0, keepdims=True)
        ex = jnp.exp(valsT - mx)
        sm = ex / jnp.sum(ex, axis=0, keepdims=True)
        out_ref[:, lo:lo + C] = sm * mask_ref[:, lo:lo + C]
        idx_ref[:, lo:lo + C] = idxT
        selacc = selacc + (work == -jnp.inf).astype(jnp.float32)

    cnt = jnp.sum(selacc, axis=1, keepdims=True)

    @pl.when(i == 0)
    def _init():
        cnt_ref[...] = jnp.zeros_like(cnt_ref)

    cnt_ref[...] += cnt

    @pl.when(i == nsteps - 1)
    def _finish():
        c_avg = jnp.float32(N) / jnp.float32(E)
        e_i = c_avg - cnt_ref[...]
        nbi_ref[...] = bi_ref[...] + LOAD_LR * jnp.sign(e_i)


def kernel(x, W, b, bi):
    mask = jnp.asarray(_MASK_NP.T)

    grid = (N // T,)
    out, idx, nbi = pl.pallas_call(
        _router_kernel,
        grid=grid,
        in_specs=[
            pl.BlockSpec((T, DIM), lambda i: (i, 0)),
            pl.BlockSpec((E, DIM), lambda i: (0, 0)),
            pl.BlockSpec((E, 1), lambda i: (0, 0)),
            pl.BlockSpec((E, 1), lambda i: (0, 0)),
            pl.BlockSpec((K, T), lambda i: (0, i)),
        ],
        out_specs=[
            pl.BlockSpec((K, T), lambda i: (0, i)),
            pl.BlockSpec((K, T), lambda i: (0, i)),
            pl.BlockSpec((E, 1), lambda i: (0, 0)),
        ],
        out_shape=[
            jax.ShapeDtypeStruct((K, N), jnp.float32),
            jax.ShapeDtypeStruct((K, N), jnp.int32),
            jax.ShapeDtypeStruct((E, 1), jnp.float32),
        ],
        scratch_shapes=[pltpu.VMEM((E, 1), jnp.float32)],
    )(x, W, b.reshape(E, 1), bi.reshape(E, 1), mask)

    return out.T, idx.T, nbi.reshape(E)


# speedup vs baseline: 1.6072x; 1.6072x over previous
import functools

import jax
import jax.numpy as jnp
import numpy as np
from jax.experimental import pallas as pl
from jax.experimental.pallas import tpu as pltpu

DIM = 768
E = 64
K = 8
FILTER_RADIO = 0.62
LOAD_LR = 0.001
N = 32768
T = 4096
C = 256


def _threefry2x32(ks0, ks1, x0, x1):
    rot_a = (13, 15, 26, 6)
    rot_b = (17, 29, 16, 24)
    ks2 = np.uint32(ks0 ^ ks1 ^ np.uint32(0x1BD11BDA))
    ks = (np.uint32(ks0), np.uint32(ks1), ks2)
    x0 = (x0 + ks[0]).astype(np.uint32)
    x1 = (x1 + ks[1]).astype(np.uint32)

    def rotl(v, d):
        return ((v << np.uint32(d)) | (v >> np.uint32(32 - d))).astype(np.uint32)

    def rounds(x0, x1, rots):
        for r in rots:
            x0 = (x0 + x1).astype(np.uint32)
            x1 = x0 ^ rotl(x1, r)
        return x0, x1

    x0, x1 = rounds(x0, x1, rot_a)
    x0 = (x0 + ks[1]).astype(np.uint32)
    x1 = (x1 + ks[2] + np.uint32(1)).astype(np.uint32)
    x0, x1 = rounds(x0, x1, rot_b)
    x0 = (x0 + ks[2]).astype(np.uint32)
    x1 = (x1 + ks[0] + np.uint32(2)).astype(np.uint32)
    x0, x1 = rounds(x0, x1, rot_a)
    x0 = (x0 + ks[0]).astype(np.uint32)
    x1 = (x1 + ks[1] + np.uint32(3)).astype(np.uint32)
    x0, x1 = rounds(x0, x1, rot_b)
    x0 = (x0 + ks[1]).astype(np.uint32)
    x1 = (x1 + ks[2] + np.uint32(4)).astype(np.uint32)
    x0, x1 = rounds(x0, x1, rot_a)
    x0 = (x0 + ks[2]).astype(np.uint32)
    x1 = (x1 + ks[0] + np.uint32(5)).astype(np.uint32)
    return x0, x1


def _compute_mask_const() -> np.ndarray:
    with np.errstate(over="ignore"):
        k0, k1 = _threefry2x32(np.uint32(0), np.uint32(0),
                               np.uint32(0), np.uint32(123))
    size = N * K
    counts = np.arange(size, dtype=np.uint32)
    b0, b1 = _threefry2x32(k0, k1, np.zeros(size, dtype=np.uint32), counts)
    bits = b0 ^ b1
    fl = ((bits >> np.uint32(9)) | np.uint32(0x3F800000)).view(np.float32)
    u = np.maximum(np.float32(0.0), fl - np.float32(1.0))
    return (u.reshape(N, K) > np.float32(FILTER_RADIO)).astype(np.float32)


_MASK_NP = _compute_mask_const()


def _router_kernel(x_ref, W_ref, b_ref, bi_ref, mask_ref,
                   out_ref, idx_ref, nbi_ref, cnt_ref):
    i = pl.program_id(0)
    nsteps = pl.num_programs(0)

    rows = jax.lax.broadcasted_iota(jnp.int32, (E, C), 0)
    selacc = jnp.zeros((E, C), jnp.float32)
    for c in range(T // C):
        lo = c * C
        logits_c = jax.lax.dot_general(
            W_ref[...], x_ref[lo:lo + C, :], (((1,), (1,)), ((), ())),
            preferred_element_type=jnp.float32)
        logits_c = logits_c + b_ref[...]
        work = logits_c + bi_ref[...]
        vals = []
        idxs = []
        for _ in range(K):
            m = jnp.max(work, axis=0, keepdims=True)
            eq = work == m
            idx = jnp.min(jnp.where(eq, rows, E), axis=0,
                          keepdims=True)
            onehot = rows == idx
            vals.append(jnp.sum(jnp.where(onehot, logits_c, 0.0), axis=0,
                                keepdims=True))
            idxs.append(idx)
            work = jnp.where(onehot, -jnp.inf, work)

        valsT = jnp.concatenate(vals, axis=0)
        idxT = jnp.concatenate(idxs, axis=0)

        mx = jnp.max(valsT, axis=0, keepdims=True)
        ex = jnp.exp(valsT - mx)
        sm = ex / jnp.sum(ex, axis=0, keepdims=True)
        out_ref[:, lo:lo + C] = sm * mask_ref[:, lo:lo + C]
        idx_ref[:, lo:lo + C] = idxT
        selacc = selacc + (work == -jnp.inf).astype(jnp.float32)

    cnt = jnp.sum(selacc, axis=1, keepdims=True)

    @pl.when(i == 0)
    def _init():
        cnt_ref[...] = jnp.zeros_like(cnt_ref)

    cnt_ref[...] += cnt

    @pl.when(i == nsteps - 1)
    def _finish():
        c_avg = jnp.float32(N) / jnp.float32(E)
        e_i = c_avg - cnt_ref[...]
        nbi_ref[...] = bi_ref[...] + LOAD_LR * jnp.sign(e_i)


def kernel(x, W, b, bi):
    mask = jnp.asarray(_MASK_NP.T)

    grid = (N // T,)
    out, idx, nbi = pl.pallas_call(
        _router_kernel,
        grid=grid,
        in_specs=[
            pl.BlockSpec((T, DIM), lambda i: (i, 0)),
            pl.BlockSpec((E, DIM), lambda i: (0, 0)),
            pl.BlockSpec((E, 1), lambda i: (0, 0)),
            pl.BlockSpec((E, 1), lambda i: (0, 0)),
            pl.BlockSpec((K, T), lambda i: (0, i)),
        ],
        out_specs=[
            pl.BlockSpec((K, T), lambda i: (0, i)),
            pl.BlockSpec((K, T), lambda i: (0, i)),
            pl.BlockSpec((E, 1), lambda i: (0, 0)),
        ],
        out_shape=[
            jax.ShapeDtypeStruct((K, N), jnp.float32),
            jax.ShapeDtypeStruct((K, N), jnp.int32),
            jax.ShapeDtypeStruct((E, 1), jnp.float32),
        ],
        scratch_shapes=[pltpu.VMEM((E, 1), jnp.float32)],
    )(x, W, b.reshape(E, 1), bi.reshape(E, 1), mask)

    return out.T, idx.T, nbi.reshape(E)
